# Initial kernel scaffold; baseline (speedup 1.0000x reference)
import jax
import jax.numpy as jnp
from jax.experimental import pallas as pl


def kernel(x, table):
    raise NotImplementedError("write your pallas kernel here")


# speedup vs baseline: 1.4599x; 1.4599x over previous
import functools

import jax
import jax.numpy as jnp
from jax import lax
from jax.experimental import pallas as pl
from jax.experimental.pallas import tpu as pltpu
from jax.experimental.pallas import tpu_sc as plsc

NUM_CORES = 2
NUM_SUBCORES = 16
NW = NUM_CORES * NUM_SUBCORES

B = 4096 * 200
D = 32
ROWS_PER_STREAM = 128
K = 8
CHUNK = ROWS_PER_STREAM * K
B_PER_W = B // NW
N_CHUNKS = B_PER_W // CHUNK


def _emb_body(table_hbm, idx_hbm, out_hbm, idx_v, rows_v, gsem, osem):
    wid = lax.axis_index("s") * NUM_CORES + lax.axis_index("c")
    row0 = wid * (B_PER_W // ROWS_PER_STREAM)

    def chunk(g, carry):
        r0 = row0 + g * K
        pltpu.sync_copy(idx_hbm.at[pl.ds(r0, K)], idx_v)
        copies = [
            pltpu.async_copy(
                table_hbm.at[idx_v.at[j]],
                rows_v.at[pl.ds(j * ROWS_PER_STREAM, ROWS_PER_STREAM)],
                gsem,
            )
            for j in range(K)
        ]
        for c in copies:
            c.wait()
        pltpu.sync_copy(
            rows_v, out_hbm.at[pl.ds(wid * B_PER_W + g * CHUNK, CHUNK)]
        )
        return carry

    lax.fori_loop(0, N_CHUNKS, chunk, 0)


@jax.jit
def kernel(x, table):
    token_ids = x[..., 0].reshape(B // ROWS_PER_STREAM, ROWS_PER_STREAM)
    token_ids = token_ids.astype(jnp.int32)

    mesh = plsc.VectorSubcoreMesh(
        core_axis_name="c", subcore_axis_name="s",
        num_cores=NUM_CORES, num_subcores=NUM_SUBCORES,
    )
    run = pl.kernel(
        _emb_body,
        out_type=jax.ShapeDtypeStruct((B, D), jnp.float32),
        mesh=mesh,
        scratch_types=[
            pltpu.VMEM((K, ROWS_PER_STREAM), jnp.int32),
            pltpu.VMEM((CHUNK, D), jnp.float32),
            pltpu.SemaphoreType.DMA,
            pltpu.SemaphoreType.DMA,
        ],
        compiler_params=pltpu.CompilerParams(use_tc_tiling_on_sc=False),
    )
    out = run(table, token_ids)
    return out.reshape(x.shape[0], x.shape[1], D)


# speedup vs baseline: 1.4960x; 1.0247x over previous
import jax
import jax.numpy as jnp
from jax import lax
from jax.experimental import pallas as pl
from jax.experimental.pallas import tpu as pltpu
from jax.experimental.pallas import tpu_sc as plsc

NUM_CORES = 2
NUM_SUBCORES = 16
NW = NUM_CORES * NUM_SUBCORES

B = 4096 * 200
D = 32
RPS = 128
K = 10
CHUNK = RPS * K
B_PER_W = B // NW
N_CHUNKS = B_PER_W // CHUNK
N_OUTER = N_CHUNKS // 2


def _emb_body(table_hbm, idx_hbm, out_hbm,
              ib0, ib1, rb0, rb1, isem0, isem1, gsem, osem0, osem1):
    wid = lax.axis_index("s") * NUM_CORES + lax.axis_index("c")
    irow0 = wid * (B_PER_W // RPS)
    out0 = wid * B_PER_W

    def idx_cp(g, ib, sem):
        return pltpu.make_async_copy(idx_hbm.at[pl.ds(irow0 + g * K, K)], ib, sem)

    def out_cp(g, rb, sem):
        return pltpu.make_async_copy(rb, out_hbm.at[pl.ds(out0 + g * CHUNK, CHUNK)], sem)

    def gather_cp(ib, rb, j):
        return pltpu.make_async_copy(
            table_hbm.at[ib.at[j]], rb.at[pl.ds(j * RPS, RPS)], gsem)

    idx_cp(0, ib0, isem0).start()

    def outer(i, carry):
        g0 = 2 * i

        idx_cp(g0, ib0, isem0).wait()

        @pl.when(i > 0)
        def _():
            out_cp(g0 - 2, rb0, osem0).wait()

        for j in range(K):
            gather_cp(ib0, rb0, j).start()
        idx_cp(g0 + 1, ib1, isem1).start()
        for j in range(K):
            gather_cp(ib0, rb0, j).wait()
        out_cp(g0, rb0, osem0).start()

        idx_cp(g0 + 1, ib1, isem1).wait()

        @pl.when(i > 0)
        def _():
            out_cp(g0 - 1, rb1, osem1).wait()

        for j in range(K):
            gather_cp(ib1, rb1, j).start()

        @pl.when(i < N_OUTER - 1)
        def _():
            idx_cp(g0 + 2, ib0, isem0).start()

        for j in range(K):
            gather_cp(ib1, rb1, j).wait()
        out_cp(g0 + 1, rb1, osem1).start()
        return carry

    lax.fori_loop(0, N_OUTER, outer, 0)

    out_cp(N_CHUNKS - 2, rb0, osem0).wait()
    out_cp(N_CHUNKS - 1, rb1, osem1).wait()


@jax.jit
def kernel(x, table):
    token_ids = x[..., 0].reshape(B // RPS, RPS).astype(jnp.int32)

    mesh = plsc.VectorSubcoreMesh(
        core_axis_name="c", subcore_axis_name="s",
        num_cores=NUM_CORES, num_subcores=NUM_SUBCORES,
    )
    run = pl.kernel(
        _emb_body,
        out_type=jax.ShapeDtypeStruct((B, D), jnp.float32),
        mesh=mesh,
        scratch_types=[
            pltpu.VMEM((K, RPS), jnp.int32),
            pltpu.VMEM((K, RPS), jnp.int32),
            pltpu.VMEM((CHUNK, D), jnp.float32),
            pltpu.VMEM((CHUNK, D), jnp.float32),
            pltpu.SemaphoreType.DMA,
            pltpu.SemaphoreType.DMA,
            pltpu.SemaphoreType.DMA,
            pltpu.SemaphoreType.DMA,
            pltpu.SemaphoreType.DMA,
        ],
        compiler_params=pltpu.CompilerParams(use_tc_tiling_on_sc=False),
    )
    out = run(table, token_ids)
    return out.reshape(x.shape[0], x.shape[1], D)
